# Initial kernel scaffold; baseline (speedup 1.0000x reference)
import jax
import jax.numpy as jnp
from jax.experimental import pallas as pl


def kernel(edge_index_clicks, edge_index_clicked_by, emb_user, emb_item, We_clicks, be_clicks, We_cb, be_cb, Wn_user, bn_user, Wn_item, bn_item):
    raise NotImplementedError("write your pallas kernel here")


# speedup vs baseline: 48.5171x; 48.5171x over previous
import functools

import jax
import jax.numpy as jnp
from jax import lax
from jax.experimental import pallas as pl
from jax.experimental.pallas import tpu as pltpu
from jax.experimental.pallas import tpu_sc as plsc

_N = 10000
_E = 320000
_D = 128
_NW = 32
_CHUNK = _E // _NW
_LANES = 16
_N_PAD = 10240
_ROWS = 1024


def _sc_flags(dst_c, dst_b):
  mesh = plsc.VectorSubcoreMesh(core_axis_name="c", subcore_axis_name="s")

  @functools.partial(
      pl.kernel,
      mesh=mesh,
      out_type=(
          jax.ShapeDtypeStruct((_NW, _N_PAD), jnp.float32),
          jax.ShapeDtypeStruct((_NW, _N_PAD), jnp.float32),
      ),
      scratch_types=[
          pltpu.VMEM((_CHUNK,), jnp.int32),
          pltpu.VMEM((_N_PAD,), jnp.float32),
      ],
      compiler_params=pltpu.CompilerParams(needs_layout_passes=False),
  )
  def k(dst_c_hbm, dst_b_hbm, fc_hbm, fb_hbm, idx_v, flag_v):
    wid = lax.axis_index("s") * 2 + lax.axis_index("c")
    base = wid * _CHUNK
    zeros = jnp.zeros((_LANES,), jnp.float32)
    ones = jnp.ones((_LANES,), jnp.float32)

    def one_etype(dst_hbm, f_hbm):
      pltpu.sync_copy(dst_hbm.at[pl.ds(base, _CHUNK)], idx_v)

      def zero_body(i, carry):
        flag_v[pl.ds(i * _LANES, _LANES)] = zeros
        return carry

      lax.fori_loop(0, _N_PAD // _LANES, zero_body, 0)

      def scatter_body(i, carry):
        idx = idx_v[pl.ds(i * _LANES, _LANES)]
        plsc.store_scatter(flag_v, [idx], ones)
        return carry

      lax.fori_loop(0, _CHUNK // _LANES, scatter_body, 0)
      pltpu.sync_copy(flag_v, f_hbm.at[wid])

    one_etype(dst_c_hbm, fc_hbm)
    one_etype(dst_b_hbm, fb_hbm)

  return k(dst_c, dst_b)


def _tc_body(fu, fi, eu, ei, wc, bc, wb, bb, wu, bu, wi, bi, ou, oi):
  v_mc = jnp.maximum(eu[...] @ wc[...] + bc[...], 0.0)
  v_mb = jnp.maximum(ei[...] @ wb[...] + bb[...], 0.0)
  base_u = eu[...] @ wu[:_D] + bu[...]
  row_a_u = jnp.maximum(base_u + v_mb @ wu[_D:], 0.0)
  row_b_u = jnp.maximum(base_u, 0.0)
  base_i = ei[...] @ wi[:_D] + bi[...]
  row_a_i = jnp.maximum(base_i + v_mc @ wi[_D:], 0.0)
  row_b_i = jnp.maximum(base_i, 0.0)
  fu_blk = jnp.max(fu[...], axis=0)
  fi_blk = jnp.max(fi[...], axis=0)
  ou[...] = jnp.where(fu_blk[:, None] > 0.0, row_a_u, row_b_u)
  oi[...] = jnp.where(fi_blk[:, None] > 0.0, row_a_i, row_b_i)


def _tc_assemble(flags_u, flags_i, emb_u, emb_i, we_c, be_c, we_b, be_b,
                 wn_u, bn_u, wn_i, bn_i, interpret=False):
  full = lambda s: pl.BlockSpec(s, lambda j: (0,) * len(s))
  return pl.pallas_call(
      _tc_body,
      grid=(_N_PAD // _ROWS,),
      in_specs=[
          pl.BlockSpec((_NW, _ROWS), lambda j: (0, j)),
          pl.BlockSpec((_NW, _ROWS), lambda j: (0, j)),
          full((1, _D)), full((1, _D)),
          full((_D, _D)), full((1, _D)),
          full((_D, _D)), full((1, _D)),
          full((2 * _D, _D)), full((1, _D)),
          full((2 * _D, _D)), full((1, _D)),
      ],
      out_specs=[
          pl.BlockSpec((_ROWS, _D), lambda j: (j, 0)),
          pl.BlockSpec((_ROWS, _D), lambda j: (j, 0)),
      ],
      out_shape=[jax.ShapeDtypeStruct((_N_PAD, _D), jnp.float32)] * 2,
      interpret=interpret,
  )(flags_u, flags_i, emb_u, emb_i,
    we_c, be_c.reshape(1, _D), we_b, be_b.reshape(1, _D),
    wn_u, bn_u.reshape(1, _D), wn_i, bn_i.reshape(1, _D))


def kernel(edge_index_clicks, edge_index_clicked_by, emb_user, emb_item,
           We_clicks, be_clicks, We_cb, be_cb,
           Wn_user, bn_user, Wn_item, bn_item):
  dst_c = edge_index_clicks[1].astype(jnp.int32)
  dst_b = edge_index_clicked_by[1].astype(jnp.int32)
  flags_item, flags_user = _sc_flags(dst_c, dst_b)
  out_u, out_i = _tc_assemble(
      flags_user, flags_item, emb_user, emb_item,
      We_clicks, be_clicks, We_cb, be_cb,
      Wn_user, bn_user, Wn_item, bn_item)
  return out_u[:_N], out_i[:_N]


# speedup vs baseline: 54.9164x; 1.1319x over previous
import functools

import jax
import jax.numpy as jnp
from jax import lax
from jax.experimental import pallas as pl
from jax.experimental.pallas import tpu as pltpu
from jax.experimental.pallas import tpu_sc as plsc

_N = 10000
_E = 320000
_D = 128
_NW = 32
_CHUNK = _E // _NW
_LANES = 16
_N_PAD = 10240
_ROWS = 1024


def _sc_flags(dst_c, dst_b):
  mesh = plsc.VectorSubcoreMesh(core_axis_name="c", subcore_axis_name="s")

  @functools.partial(
      pl.kernel,
      mesh=mesh,
      out_type=(
          jax.ShapeDtypeStruct((_NW, _N_PAD), jnp.float32),
          jax.ShapeDtypeStruct((_NW, _N_PAD), jnp.float32),
      ),
      scratch_types=[
          pltpu.VMEM((_CHUNK,), jnp.int32),
          pltpu.VMEM((_N_PAD,), jnp.float32),
      ],
      compiler_params=pltpu.CompilerParams(needs_layout_passes=False),
  )
  def k(dst_c_hbm, dst_b_hbm, fc_hbm, fb_hbm, idx_v, flag_v):
    wid = lax.axis_index("s") * 2 + lax.axis_index("c")
    base = wid * _CHUNK
    zeros = jnp.zeros((_LANES,), jnp.float32)
    ones = jnp.ones((_LANES,), jnp.float32)

    def one_etype(dst_hbm, f_hbm):
      pltpu.sync_copy(dst_hbm.at[pl.ds(base, _CHUNK)], idx_v)

      def zero_body(i, carry):
        flag_v[pl.ds(i * _LANES, _LANES)] = zeros
        return carry

      lax.fori_loop(0, _N_PAD // _LANES, zero_body, 0)

      def scatter_body(i, carry):
        idx = idx_v[pl.ds(i * _LANES, _LANES)]
        plsc.store_scatter(flag_v, [idx], ones)
        return carry

      lax.fori_loop(0, _CHUNK // _LANES, scatter_body, 0)
      pltpu.sync_copy(flag_v, f_hbm.at[wid])

    one_etype(dst_c_hbm, fc_hbm)
    one_etype(dst_b_hbm, fb_hbm)

  return k(dst_c, dst_b)


def _tc_body(fu, fi, eu, ei, wc, bc, wb, bb, wu, bu, wi, bi, ou, oi):
  v_mc = jnp.maximum(eu[...] @ wc[...] + bc[...], 0.0)
  v_mb = jnp.maximum(ei[...] @ wb[...] + bb[...], 0.0)
  base_u = eu[...] @ wu[:_D] + bu[...]
  row_a_u = jnp.maximum(base_u + v_mb @ wu[_D:], 0.0)
  row_b_u = jnp.maximum(base_u, 0.0)
  base_i = ei[...] @ wi[:_D] + bi[...]
  row_a_i = jnp.maximum(base_i + v_mc @ wi[_D:], 0.0)
  row_b_i = jnp.maximum(base_i, 0.0)
  fu_blk = jnp.max(fu[...], axis=0)
  fi_blk = jnp.max(fi[...], axis=0)
  ou[...] = jnp.where(fu_blk[:, None] > 0.0, row_a_u, row_b_u)
  oi[...] = jnp.where(fi_blk[:, None] > 0.0, row_a_i, row_b_i)


def _tc_assemble(flags_u, flags_i, emb_u, emb_i, we_c, be_c, we_b, be_b,
                 wn_u, bn_u, wn_i, bn_i, interpret=False):
  full = lambda s: pl.BlockSpec(s, lambda j: (0,) * len(s))
  return pl.pallas_call(
      _tc_body,
      grid=(_N_PAD // _ROWS,),
      in_specs=[
          pl.BlockSpec((_NW, _ROWS), lambda j: (0, j)),
          pl.BlockSpec((_NW, _ROWS), lambda j: (0, j)),
          full((1, _D)), full((1, _D)),
          full((_D, _D)), full((1, _D)),
          full((_D, _D)), full((1, _D)),
          full((2 * _D, _D)), full((1, _D)),
          full((2 * _D, _D)), full((1, _D)),
      ],
      out_specs=[
          pl.BlockSpec((_ROWS, _D), lambda j: (j, 0)),
          pl.BlockSpec((_ROWS, _D), lambda j: (j, 0)),
      ],
      out_shape=[jax.ShapeDtypeStruct((_N, _D), jnp.float32)] * 2,
      interpret=interpret,
  )(flags_u, flags_i, emb_u, emb_i,
    we_c, be_c.reshape(1, _D), we_b, be_b.reshape(1, _D),
    wn_u, bn_u.reshape(1, _D), wn_i, bn_i.reshape(1, _D))


def kernel(edge_index_clicks, edge_index_clicked_by, emb_user, emb_item,
           We_clicks, be_clicks, We_cb, be_cb,
           Wn_user, bn_user, Wn_item, bn_item):
  dst_c = edge_index_clicks[1].astype(jnp.int32)
  dst_b = edge_index_clicked_by[1].astype(jnp.int32)
  flags_item, flags_user = _sc_flags(dst_c, dst_b)
  out_u, out_i = _tc_assemble(
      flags_user, flags_item, emb_user, emb_item,
      We_clicks, be_clicks, We_cb, be_cb,
      Wn_user, bn_user, Wn_item, bn_item)
  return out_u, out_i


# speedup vs baseline: 56.6596x; 1.0317x over previous
import functools

import jax
import jax.numpy as jnp
from jax import lax
from jax.experimental import pallas as pl
from jax.experimental.pallas import tpu as pltpu
from jax.experimental.pallas import tpu_sc as plsc

_N = 10000
_E = 320000
_D = 128
_NW = 32
_CHUNK = _E // _NW
_LANES = 16
_N_PAD = 10240
_ROWS = 1024


def _sc_flags(dst_c, dst_b, zeros2):
  mesh = plsc.VectorSubcoreMesh(core_axis_name="c", subcore_axis_name="s")

  @functools.partial(
      pl.kernel,
      mesh=mesh,
      out_type=(
          jax.ShapeDtypeStruct((_NW, _N_PAD), jnp.float32),
          jax.ShapeDtypeStruct((_NW, _N_PAD), jnp.float32),
      ),
      scratch_types=[
          pltpu.VMEM((_CHUNK,), jnp.int32),
          pltpu.VMEM((_CHUNK,), jnp.int32),
          pltpu.VMEM((2 * _N_PAD,), jnp.float32),
          pltpu.SemaphoreType.DMA,
          pltpu.SemaphoreType.DMA,
      ],
      compiler_params=pltpu.CompilerParams(needs_layout_passes=False),
  )
  def k(dst_c_hbm, dst_b_hbm, zeros_hbm, fc_hbm, fb_hbm,
        idx_c, idx_b, flag_v, sem_c, sem_b):
    wid = lax.axis_index("s") * 2 + lax.axis_index("c")
    base = wid * _CHUNK
    ones = jnp.ones((_LANES,), jnp.float32)
    off = jnp.full((_LANES,), _N_PAD, jnp.int32)

    cp_c = pltpu.async_copy(dst_c_hbm.at[pl.ds(base, _CHUNK)], idx_c, sem_c)
    cp_b = pltpu.async_copy(dst_b_hbm.at[pl.ds(base, _CHUNK)], idx_b, sem_b)
    pltpu.sync_copy(zeros_hbm, flag_v)
    cp_c.wait()

    def body_c(i, carry):
      idx = idx_c[pl.ds(i * _LANES, _LANES)]
      plsc.store_scatter(flag_v, [idx], ones)
      return carry

    lax.fori_loop(0, _CHUNK // _LANES, body_c, 0)
    cp_b.wait()

    def body_b(i, carry):
      idx = idx_b[pl.ds(i * _LANES, _LANES)] + off
      plsc.store_scatter(flag_v, [idx], ones)
      return carry

    lax.fori_loop(0, _CHUNK // _LANES, body_b, 0)

    pltpu.sync_copy(flag_v.at[pl.ds(0, _N_PAD)], fc_hbm.at[wid])
    pltpu.sync_copy(flag_v.at[pl.ds(_N_PAD, _N_PAD)], fb_hbm.at[wid])

  return k(dst_c, dst_b, zeros2)


def _tc_body(fu, fi, eu, ei, wc, bc, wb, bb, wu, bu, wi, bi, ou, oi):
  v_mc = jnp.maximum(eu[...] @ wc[...] + bc[...], 0.0)
  v_mb = jnp.maximum(ei[...] @ wb[...] + bb[...], 0.0)
  base_u = eu[...] @ wu[:_D] + bu[...]
  row_a_u = jnp.maximum(base_u + v_mb @ wu[_D:], 0.0)
  row_b_u = jnp.maximum(base_u, 0.0)
  base_i = ei[...] @ wi[:_D] + bi[...]
  row_a_i = jnp.maximum(base_i + v_mc @ wi[_D:], 0.0)
  row_b_i = jnp.maximum(base_i, 0.0)
  fu_blk = jnp.max(fu[...], axis=0)
  fi_blk = jnp.max(fi[...], axis=0)
  ou[...] = jnp.where(fu_blk[:, None] > 0.0, row_a_u, row_b_u)
  oi[...] = jnp.where(fi_blk[:, None] > 0.0, row_a_i, row_b_i)


def _tc_assemble(flags_u, flags_i, emb_u, emb_i, we_c, be_c, we_b, be_b,
                 wn_u, bn_u, wn_i, bn_i, interpret=False):
  full = lambda s: pl.BlockSpec(s, lambda j: (0,) * len(s))
  return pl.pallas_call(
      _tc_body,
      grid=(_N_PAD // _ROWS,),
      in_specs=[
          pl.BlockSpec((_NW, _ROWS), lambda j: (0, j)),
          pl.BlockSpec((_NW, _ROWS), lambda j: (0, j)),
          full((1, _D)), full((1, _D)),
          full((_D, _D)), full((1, _D)),
          full((_D, _D)), full((1, _D)),
          full((2 * _D, _D)), full((1, _D)),
          full((2 * _D, _D)), full((1, _D)),
      ],
      out_specs=[
          pl.BlockSpec((_ROWS, _D), lambda j: (j, 0)),
          pl.BlockSpec((_ROWS, _D), lambda j: (j, 0)),
      ],
      out_shape=[jax.ShapeDtypeStruct((_N, _D), jnp.float32)] * 2,
      interpret=interpret,
  )(flags_u, flags_i, emb_u, emb_i,
    we_c, be_c.reshape(1, _D), we_b, be_b.reshape(1, _D),
    wn_u, bn_u.reshape(1, _D), wn_i, bn_i.reshape(1, _D))


def kernel(edge_index_clicks, edge_index_clicked_by, emb_user, emb_item,
           We_clicks, be_clicks, We_cb, be_cb,
           Wn_user, bn_user, Wn_item, bn_item):
  dst_c = edge_index_clicks[1].astype(jnp.int32)
  dst_b = edge_index_clicked_by[1].astype(jnp.int32)
  zeros2 = jnp.zeros((2 * _N_PAD,), jnp.float32)
  flags_item, flags_user = _sc_flags(dst_c, dst_b, zeros2)
  out_u, out_i = _tc_assemble(
      flags_user, flags_item, emb_user, emb_item,
      We_clicks, be_clicks, We_cb, be_cb,
      Wn_user, bn_user, Wn_item, bn_item)
  return out_u, out_i


# speedup vs baseline: 57.3984x; 1.0130x over previous
import functools

import jax
import jax.numpy as jnp
from jax import lax
from jax.experimental import pallas as pl
from jax.experimental.pallas import tpu as pltpu
from jax.experimental.pallas import tpu_sc as plsc

_N = 10000
_E = 320000
_D = 128
_NW = 32
_CHUNK = _E // _NW
_LANES = 16
_N_PAD = 10240
_ROWS = 1024


def _sc_flags(dst_c, dst_b, zeros2):
  mesh = plsc.VectorSubcoreMesh(core_axis_name="c", subcore_axis_name="s")

  @functools.partial(
      pl.kernel,
      mesh=mesh,
      out_type=(
          jax.ShapeDtypeStruct((_NW, _N_PAD), jnp.float32),
          jax.ShapeDtypeStruct((_NW, _N_PAD), jnp.float32),
      ),
      scratch_types=[
          pltpu.VMEM((_CHUNK,), jnp.int32),
          pltpu.VMEM((_CHUNK,), jnp.int32),
          pltpu.VMEM((2 * _N_PAD,), jnp.float32),
          pltpu.SemaphoreType.DMA,
          pltpu.SemaphoreType.DMA,
      ],
      compiler_params=pltpu.CompilerParams(needs_layout_passes=False),
  )
  def k(dst_c_hbm, dst_b_hbm, zeros_hbm, fc_hbm, fb_hbm,
        idx_c, idx_b, flag_v, sem_c, sem_b):
    wid = lax.axis_index("s") * 2 + lax.axis_index("c")
    base = wid * _CHUNK
    ones = jnp.ones((_LANES,), jnp.float32)
    off = jnp.full((_LANES,), _N_PAD, jnp.int32)

    cp_c = pltpu.async_copy(dst_c_hbm.at[pl.ds(base, _CHUNK)], idx_c, sem_c)
    cp_b = pltpu.async_copy(dst_b_hbm.at[pl.ds(base, _CHUNK)], idx_b, sem_b)
    pltpu.sync_copy(zeros_hbm, flag_v)
    cp_c.wait()

    unroll = 5
    n_outer = _CHUNK // _LANES // unroll

    def body_c(i, carry):
      for j in range(unroll):
        idx = idx_c[pl.ds((i * unroll + j) * _LANES, _LANES)]
        plsc.store_scatter(flag_v, [idx], ones)
      return carry

    lax.fori_loop(0, n_outer, body_c, 0)
    cp_b.wait()

    def body_b(i, carry):
      for j in range(unroll):
        idx = idx_b[pl.ds((i * unroll + j) * _LANES, _LANES)] + off
        plsc.store_scatter(flag_v, [idx], ones)
      return carry

    lax.fori_loop(0, n_outer, body_b, 0)

    pltpu.sync_copy(flag_v.at[pl.ds(0, _N_PAD)], fc_hbm.at[wid])
    pltpu.sync_copy(flag_v.at[pl.ds(_N_PAD, _N_PAD)], fb_hbm.at[wid])

  return k(dst_c, dst_b, zeros2)


def _tc_body(fu, fi, eu, ei, wc, bc, wb, bb, wu, bu, wi, bi, ou, oi):
  v_mc = jnp.maximum(eu[...] @ wc[...] + bc[...], 0.0)
  v_mb = jnp.maximum(ei[...] @ wb[...] + bb[...], 0.0)
  base_u = eu[...] @ wu[:_D] + bu[...]
  row_a_u = jnp.maximum(base_u + v_mb @ wu[_D:], 0.0)
  row_b_u = jnp.maximum(base_u, 0.0)
  base_i = ei[...] @ wi[:_D] + bi[...]
  row_a_i = jnp.maximum(base_i + v_mc @ wi[_D:], 0.0)
  row_b_i = jnp.maximum(base_i, 0.0)
  fu_blk = jnp.max(fu[...], axis=0)
  fi_blk = jnp.max(fi[...], axis=0)
  ou[...] = jnp.where(fu_blk[:, None] > 0.0, row_a_u, row_b_u)
  oi[...] = jnp.where(fi_blk[:, None] > 0.0, row_a_i, row_b_i)


def _tc_assemble(flags_u, flags_i, emb_u, emb_i, we_c, be_c, we_b, be_b,
                 wn_u, bn_u, wn_i, bn_i, interpret=False):
  full = lambda s: pl.BlockSpec(s, lambda j: (0,) * len(s))
  return pl.pallas_call(
      _tc_body,
      grid=(_N_PAD // _ROWS,),
      in_specs=[
          pl.BlockSpec((_NW, _ROWS), lambda j: (0, j)),
          pl.BlockSpec((_NW, _ROWS), lambda j: (0, j)),
          full((1, _D)), full((1, _D)),
          full((_D, _D)), full((1, _D)),
          full((_D, _D)), full((1, _D)),
          full((2 * _D, _D)), full((1, _D)),
          full((2 * _D, _D)), full((1, _D)),
      ],
      out_specs=[
          pl.BlockSpec((_ROWS, _D), lambda j: (j, 0)),
          pl.BlockSpec((_ROWS, _D), lambda j: (j, 0)),
      ],
      out_shape=[jax.ShapeDtypeStruct((_N, _D), jnp.float32)] * 2,
      interpret=interpret,
  )(flags_u, flags_i, emb_u, emb_i,
    we_c, be_c.reshape(1, _D), we_b, be_b.reshape(1, _D),
    wn_u, bn_u.reshape(1, _D), wn_i, bn_i.reshape(1, _D))


def kernel(edge_index_clicks, edge_index_clicked_by, emb_user, emb_item,
           We_clicks, be_clicks, We_cb, be_cb,
           Wn_user, bn_user, Wn_item, bn_item):
  dst_c = edge_index_clicks[1].astype(jnp.int32)
  dst_b = edge_index_clicked_by[1].astype(jnp.int32)
  zeros2 = jnp.zeros((2 * _N_PAD,), jnp.float32)
  flags_item, flags_user = _sc_flags(dst_c, dst_b, zeros2)
  out_u, out_i = _tc_assemble(
      flags_user, flags_item, emb_user, emb_item,
      We_clicks, be_clicks, We_cb, be_cb,
      Wn_user, bn_user, Wn_item, bn_item)
  return out_u, out_i


# speedup vs baseline: 71.4581x; 1.2449x over previous
import functools

import jax
import jax.numpy as jnp
from jax import lax
from jax.experimental import pallas as pl
from jax.experimental.pallas import tpu as pltpu
from jax.experimental.pallas import tpu_sc as plsc

_N = 10000
_E = 320000
_D = 128
_NW = 32
_CHUNK = _E // _NW
_LANES = 16
_N_PAD = 10240
_ROWS = 1024


def _sc_flags(eic, eib):
  mesh = plsc.VectorSubcoreMesh(core_axis_name="c", subcore_axis_name="s")

  @functools.partial(
      pl.kernel,
      mesh=mesh,
      out_type=(
          jax.ShapeDtypeStruct((_NW, _N_PAD), jnp.float32),
          jax.ShapeDtypeStruct((_NW, _N_PAD), jnp.float32),
      ),
      scratch_types=[
          pltpu.VMEM((_CHUNK,), jnp.int32),
          pltpu.VMEM((_CHUNK,), jnp.int32),
          pltpu.VMEM((2 * _N_PAD,), jnp.float32),
          pltpu.SemaphoreType.DMA,
          pltpu.SemaphoreType.DMA,
      ],
      compiler_params=pltpu.CompilerParams(needs_layout_passes=False),
  )
  def k(eic_hbm, eib_hbm, fc_hbm, fb_hbm,
        idx_c, idx_b, flag_v, sem_c, sem_b):
    wid = lax.axis_index("s") * 2 + lax.axis_index("c")
    base = wid * _CHUNK
    ones = jnp.ones((_LANES,), jnp.float32)
    zeros = jnp.zeros((_LANES,), jnp.float32)
    off = jnp.full((_LANES,), _N_PAD, jnp.int32)

    cp_c = pltpu.async_copy(eic_hbm.at[pl.ds(_E + base, _CHUNK)], idx_c, sem_c)
    cp_b = pltpu.async_copy(eib_hbm.at[pl.ds(_E + base, _CHUNK)], idx_b, sem_b)

    def zero_body(i, carry):
      for j in range(8):
        flag_v[pl.ds((i * 8 + j) * _LANES, _LANES)] = zeros
      return carry

    lax.fori_loop(0, 2 * _N_PAD // _LANES // 8, zero_body, 0)
    cp_c.wait()

    unroll = 5
    n_outer = _CHUNK // _LANES // unroll

    def body_c(i, carry):
      for j in range(unroll):
        idx = idx_c[pl.ds((i * unroll + j) * _LANES, _LANES)]
        plsc.store_scatter(flag_v, [idx], ones)
      return carry

    lax.fori_loop(0, n_outer, body_c, 0)
    cp_b.wait()

    def body_b(i, carry):
      for j in range(unroll):
        idx = idx_b[pl.ds((i * unroll + j) * _LANES, _LANES)] + off
        plsc.store_scatter(flag_v, [idx], ones)
      return carry

    lax.fori_loop(0, n_outer, body_b, 0)

    pltpu.sync_copy(flag_v.at[pl.ds(0, _N_PAD)], fc_hbm.at[wid])
    pltpu.sync_copy(flag_v.at[pl.ds(_N_PAD, _N_PAD)], fb_hbm.at[wid])

  return k(eic, eib)


def _tc_body(fu, fi, eu, ei, wc, bc, wb, bb, wu, bu, wi, bi, ou, oi):
  v_mc = jnp.maximum(eu[...] @ wc[...] + bc[...], 0.0)
  v_mb = jnp.maximum(ei[...] @ wb[...] + bb[...], 0.0)
  base_u = eu[...] @ wu[:_D] + bu[...]
  row_a_u = jnp.maximum(base_u + v_mb @ wu[_D:], 0.0)
  row_b_u = jnp.maximum(base_u, 0.0)
  base_i = ei[...] @ wi[:_D] + bi[...]
  row_a_i = jnp.maximum(base_i + v_mc @ wi[_D:], 0.0)
  row_b_i = jnp.maximum(base_i, 0.0)
  fu_blk = jnp.max(fu[...], axis=0)
  fi_blk = jnp.max(fi[...], axis=0)
  ou[...] = jnp.where(fu_blk[:, None] > 0.0, row_a_u, row_b_u)
  oi[...] = jnp.where(fi_blk[:, None] > 0.0, row_a_i, row_b_i)


def _tc_assemble(flags_u, flags_i, emb_u, emb_i, we_c, be_c, we_b, be_b,
                 wn_u, bn_u, wn_i, bn_i, interpret=False):
  full = lambda s: pl.BlockSpec(s, lambda j: (0,) * len(s))
  return pl.pallas_call(
      _tc_body,
      grid=(_N_PAD // _ROWS,),
      in_specs=[
          pl.BlockSpec((_NW, _ROWS), lambda j: (0, j)),
          pl.BlockSpec((_NW, _ROWS), lambda j: (0, j)),
          full((1, _D)), full((1, _D)),
          full((_D, _D)), full((1, _D)),
          full((_D, _D)), full((1, _D)),
          full((2 * _D, _D)), full((1, _D)),
          full((2 * _D, _D)), full((1, _D)),
      ],
      out_specs=[
          pl.BlockSpec((_ROWS, _D), lambda j: (j, 0)),
          pl.BlockSpec((_ROWS, _D), lambda j: (j, 0)),
      ],
      out_shape=[jax.ShapeDtypeStruct((_N, _D), jnp.float32)] * 2,
      interpret=interpret,
  )(flags_u, flags_i, emb_u, emb_i,
    we_c, be_c.reshape(1, _D), we_b, be_b.reshape(1, _D),
    wn_u, bn_u.reshape(1, _D), wn_i, bn_i.reshape(1, _D))


def kernel(edge_index_clicks, edge_index_clicked_by, emb_user, emb_item,
           We_clicks, be_clicks, We_cb, be_cb,
           Wn_user, bn_user, Wn_item, bn_item):
  eic = edge_index_clicks.astype(jnp.int32).reshape(2 * _E)
  eib = edge_index_clicked_by.astype(jnp.int32).reshape(2 * _E)
  flags_item, flags_user = _sc_flags(eic, eib)
  out_u, out_i = _tc_assemble(
      flags_user, flags_item, emb_user, emb_item,
      We_clicks, be_clicks, We_cb, be_cb,
      Wn_user, bn_user, Wn_item, bn_item)
  return out_u, out_i


# speedup vs baseline: 85.7998x; 1.2007x over previous
import functools

import jax
import jax.numpy as jnp
from jax import lax
from jax.experimental import pallas as pl
from jax.experimental.pallas import tpu as pltpu
from jax.experimental.pallas import tpu_sc as plsc

_N = 10000
_E = 320000
_D = 128
_NW = 32
_CHUNK = _E // _NW
_LANES = 16
_N_PAD = 10240
_ROWS = 2560
_WIN = 10112


def _sc_flags(eic, eib):
  mesh = plsc.VectorSubcoreMesh(core_axis_name="c", subcore_axis_name="s")

  @functools.partial(
      pl.kernel,
      mesh=mesh,
      out_type=(
          jax.ShapeDtypeStruct((_NW, _N_PAD), jnp.float32),
          jax.ShapeDtypeStruct((_NW, _N_PAD), jnp.float32),
      ),
      scratch_types=[
          pltpu.VMEM((2, _WIN), jnp.int32),
          pltpu.VMEM((2, _WIN), jnp.int32),
          pltpu.VMEM((2 * _N_PAD,), jnp.float32),
          pltpu.SemaphoreType.DMA,
          pltpu.SemaphoreType.DMA,
      ],
      compiler_params=pltpu.CompilerParams(needs_layout_passes=False),
  )
  def k(eic_hbm, eib_hbm, fc_hbm, fb_hbm,
        idx_c, idx_b, flag_v, sem_c, sem_b):
    wid = lax.axis_index("s") * 2 + lax.axis_index("c")
    base = wid * _CHUNK
    s = lax.rem(base, 128)
    astart = pl.multiple_of(base - s, 128)
    ones = jnp.ones((_LANES,), jnp.float32)
    zeros = jnp.zeros((_LANES,), jnp.float32)
    off = jnp.full((_LANES,), _N_PAD, jnp.int32)

    cp_c = pltpu.async_copy(eic_hbm.at[:, pl.ds(astart, _WIN)], idx_c, sem_c)
    cp_b = pltpu.async_copy(eib_hbm.at[:, pl.ds(astart, _WIN)], idx_b, sem_b)

    def zero_body(i, carry):
      for j in range(8):
        flag_v[pl.ds((i * 8 + j) * _LANES, _LANES)] = zeros
      return carry

    lax.fori_loop(0, 2 * _N_PAD // _LANES // 8, zero_body, 0)
    cp_c.wait()

    unroll = 5
    n_outer = _CHUNK // _LANES // unroll

    def body_c(i, carry):
      for j in range(unroll):
        idx = idx_c[1, pl.ds(s + (i * unroll + j) * _LANES, _LANES)]
        plsc.store_scatter(flag_v, [idx], ones)
      return carry

    lax.fori_loop(0, n_outer, body_c, 0)
    cp_b.wait()

    def body_b(i, carry):
      for j in range(unroll):
        idx = idx_b[1, pl.ds(s + (i * unroll + j) * _LANES, _LANES)] + off
        plsc.store_scatter(flag_v, [idx], ones)
      return carry

    lax.fori_loop(0, n_outer, body_b, 0)

    pltpu.sync_copy(flag_v.at[pl.ds(0, _N_PAD)], fc_hbm.at[wid])
    pltpu.sync_copy(flag_v.at[pl.ds(_N_PAD, _N_PAD)], fb_hbm.at[wid])

  return k(eic, eib)


def _tc_body(fu, fi, eu, ei, wc, bc, wb, bb, wu, bu, wi, bi, ou, oi):
  v_mc = jnp.maximum(eu[...] @ wc[...] + bc[...], 0.0)
  v_mb = jnp.maximum(ei[...] @ wb[...] + bb[...], 0.0)
  base_u = eu[...] @ wu[:_D] + bu[...]
  row_a_u = jnp.maximum(base_u + v_mb @ wu[_D:], 0.0)
  row_b_u = jnp.maximum(base_u, 0.0)
  base_i = ei[...] @ wi[:_D] + bi[...]
  row_a_i = jnp.maximum(base_i + v_mc @ wi[_D:], 0.0)
  row_b_i = jnp.maximum(base_i, 0.0)
  fu_blk = jnp.max(fu[...], axis=0)
  fi_blk = jnp.max(fi[...], axis=0)
  ou[...] = jnp.where(fu_blk[:, None] > 0.0, row_a_u, row_b_u)
  oi[...] = jnp.where(fi_blk[:, None] > 0.0, row_a_i, row_b_i)


def _tc_assemble(flags_u, flags_i, emb_u, emb_i, we_c, be_c, we_b, be_b,
                 wn_u, bn_u, wn_i, bn_i, interpret=False):
  full = lambda s: pl.BlockSpec(s, lambda j: (0,) * len(s))
  return pl.pallas_call(
      _tc_body,
      grid=(_N_PAD // _ROWS,),
      in_specs=[
          pl.BlockSpec((_NW, _ROWS), lambda j: (0, j)),
          pl.BlockSpec((_NW, _ROWS), lambda j: (0, j)),
          full((1, _D)), full((1, _D)),
          full((_D, _D)), full((1, _D)),
          full((_D, _D)), full((1, _D)),
          full((2 * _D, _D)), full((1, _D)),
          full((2 * _D, _D)), full((1, _D)),
      ],
      out_specs=[
          pl.BlockSpec((_ROWS, _D), lambda j: (j, 0)),
          pl.BlockSpec((_ROWS, _D), lambda j: (j, 0)),
      ],
      out_shape=[jax.ShapeDtypeStruct((_N, _D), jnp.float32)] * 2,
      interpret=interpret,
  )(flags_u, flags_i, emb_u, emb_i,
    we_c, be_c.reshape(1, _D), we_b, be_b.reshape(1, _D),
    wn_u, bn_u.reshape(1, _D), wn_i, bn_i.reshape(1, _D))


def kernel(edge_index_clicks, edge_index_clicked_by, emb_user, emb_item,
           We_clicks, be_clicks, We_cb, be_cb,
           Wn_user, bn_user, Wn_item, bn_item):
  eic = edge_index_clicks.astype(jnp.int32)
  eib = edge_index_clicked_by.astype(jnp.int32)
  flags_item, flags_user = _sc_flags(eic, eib)
  out_u, out_i = _tc_assemble(
      flags_user, flags_item, emb_user, emb_item,
      We_clicks, be_clicks, We_cb, be_cb,
      Wn_user, bn_user, Wn_item, bn_item)
  return out_u, out_i


# speedup vs baseline: 88.8063x; 1.0350x over previous
import functools

import jax
import jax.numpy as jnp
from jax import lax
from jax.experimental import pallas as pl
from jax.experimental.pallas import tpu as pltpu
from jax.experimental.pallas import tpu_sc as plsc

_N = 10000
_E = 320000
_D = 128
_NW = 32
_CHUNK = _E // _NW
_LANES = 16
_N_PAD = 10240
_ROWS = 5120
_WIN = 10112


def _sc_flags(eic, eib):
  mesh = plsc.VectorSubcoreMesh(core_axis_name="c", subcore_axis_name="s")

  @functools.partial(
      pl.kernel,
      mesh=mesh,
      out_type=(
          jax.ShapeDtypeStruct((_NW, _N_PAD), jnp.float32),
          jax.ShapeDtypeStruct((_NW, _N_PAD), jnp.float32),
      ),
      scratch_types=[
          pltpu.VMEM((2, _WIN), jnp.int32),
          pltpu.VMEM((2, _WIN), jnp.int32),
          pltpu.VMEM((2 * _N_PAD,), jnp.float32),
          pltpu.SemaphoreType.DMA,
          pltpu.SemaphoreType.DMA,
      ],
      compiler_params=pltpu.CompilerParams(needs_layout_passes=False),
  )
  def k(eic_hbm, eib_hbm, fc_hbm, fb_hbm,
        idx_c, idx_b, flag_v, sem_c, sem_b):
    wid = lax.axis_index("s") * 2 + lax.axis_index("c")
    base = wid * _CHUNK
    s = lax.rem(base, 128)
    astart = pl.multiple_of(base - s, 128)
    ones = jnp.ones((_LANES,), jnp.float32)
    zeros = jnp.zeros((_LANES,), jnp.float32)
    off = jnp.full((_LANES,), _N_PAD, jnp.int32)

    cp_c = pltpu.async_copy(eic_hbm.at[:, pl.ds(astart, _WIN)], idx_c, sem_c)
    cp_b = pltpu.async_copy(eib_hbm.at[:, pl.ds(astart, _WIN)], idx_b, sem_b)

    def zero_body(i, carry):
      for j in range(8):
        flag_v[pl.ds((i * 8 + j) * _LANES, _LANES)] = zeros
      return carry

    lax.fori_loop(0, 2 * _N_PAD // _LANES // 8, zero_body, 0)
    cp_c.wait()
    cp_b.wait()

    unroll = 5
    n_outer = _CHUNK // _LANES // unroll

    def body(i, carry):
      for j in range(unroll):
        o = pl.ds(s + (i * unroll + j) * _LANES, _LANES)
        plsc.store_scatter(flag_v, [idx_c[1, o]], ones)
        plsc.store_scatter(flag_v, [idx_b[1, o] + off], ones)
      return carry

    lax.fori_loop(0, n_outer, body, 0)

    pltpu.sync_copy(flag_v.at[pl.ds(0, _N_PAD)], fc_hbm.at[wid])
    pltpu.sync_copy(flag_v.at[pl.ds(_N_PAD, _N_PAD)], fb_hbm.at[wid])

  return k(eic, eib)


def _tc_body(fu, fi, eu, ei, wc, bc, wb, bb, wu, bu, wi, bi, ou, oi):
  v_mc = jnp.maximum(eu[...] @ wc[...] + bc[...], 0.0)
  v_mb = jnp.maximum(ei[...] @ wb[...] + bb[...], 0.0)
  base_u = eu[...] @ wu[:_D] + bu[...]
  row_a_u = jnp.maximum(base_u + v_mb @ wu[_D:], 0.0)
  row_b_u = jnp.maximum(base_u, 0.0)
  base_i = ei[...] @ wi[:_D] + bi[...]
  row_a_i = jnp.maximum(base_i + v_mc @ wi[_D:], 0.0)
  row_b_i = jnp.maximum(base_i, 0.0)
  fu_blk = jnp.max(fu[...], axis=0)
  fi_blk = jnp.max(fi[...], axis=0)
  ou[...] = jnp.where(fu_blk[:, None] > 0.0, row_a_u, row_b_u)
  oi[...] = jnp.where(fi_blk[:, None] > 0.0, row_a_i, row_b_i)


def _tc_assemble(flags_u, flags_i, emb_u, emb_i, we_c, be_c, we_b, be_b,
                 wn_u, bn_u, wn_i, bn_i, interpret=False):
  full = lambda s: pl.BlockSpec(s, lambda j: (0,) * len(s))
  return pl.pallas_call(
      _tc_body,
      grid=(_N_PAD // _ROWS,),
      in_specs=[
          pl.BlockSpec((_NW, _ROWS), lambda j: (0, j)),
          pl.BlockSpec((_NW, _ROWS), lambda j: (0, j)),
          full((1, _D)), full((1, _D)),
          full((_D, _D)), full((1, _D)),
          full((_D, _D)), full((1, _D)),
          full((2 * _D, _D)), full((1, _D)),
          full((2 * _D, _D)), full((1, _D)),
      ],
      out_specs=[
          pl.BlockSpec((_ROWS, _D), lambda j: (j, 0)),
          pl.BlockSpec((_ROWS, _D), lambda j: (j, 0)),
      ],
      out_shape=[jax.ShapeDtypeStruct((_N, _D), jnp.float32)] * 2,
      interpret=interpret,
  )(flags_u, flags_i, emb_u, emb_i,
    we_c, be_c.reshape(1, _D), we_b, be_b.reshape(1, _D),
    wn_u, bn_u.reshape(1, _D), wn_i, bn_i.reshape(1, _D))


def kernel(edge_index_clicks, edge_index_clicked_by, emb_user, emb_item,
           We_clicks, be_clicks, We_cb, be_cb,
           Wn_user, bn_user, Wn_item, bn_item):
  eic = edge_index_clicks.astype(jnp.int32)
  eib = edge_index_clicked_by.astype(jnp.int32)
  flags_item, flags_user = _sc_flags(eic, eib)
  out_u, out_i = _tc_assemble(
      flags_user, flags_item, emb_user, emb_item,
      We_clicks, be_clicks, We_cb, be_cb,
      Wn_user, bn_user, Wn_item, bn_item)
  return out_u, out_i
